# baseline (speedup 1.0000x reference)
import functools

import jax
import jax.numpy as jnp
from jax import lax
from jax.experimental import pallas as pl
from jax.experimental.pallas import tpu as pltpu
from jax.experimental.pallas import tpu_sc as plsc

NC, NS = 2, 16
NW = NC * NS
SPW = 128
BLK = 32
K = 10


@functools.lru_cache(maxsize=None)
def _make_gather(batch: int, hist: int, d_model: int):
    assert batch == NW * SPW
    halves = SPW // BLK
    ng = hist * halves
    nwin = ng // K
    npair = nwin // 2
    assert ng % K == 0 and nwin % 2 == 0

    mesh = plsc.VectorSubcoreMesh(core_axis_name="c", subcore_axis_name="s")

    @functools.partial(
        pl.kernel,
        mesh=mesh,
        compiler_params=pltpu.CompilerParams(
            disable_bounds_checks=True,
            disable_semaphore_checks=True,
            skip_device_barrier=True,
        ),
        out_type=jax.ShapeDtypeStruct((hist, batch, d_model), jnp.float32),
        scratch_types=[
            pltpu.VMEM((hist, SPW), jnp.int32),
            pltpu.VMEM((2, K, BLK, d_model), jnp.float32),
            pltpu.SemaphoreType.DMA,
            pltpu.SemaphoreType.DMA,
            pltpu.SemaphoreType.DMA,
        ],
    )
    def gather(table_hbm, idxt_hbm, out_hbm, idx_v, rows_v, gsem0, gsem1, ssem):
        wid = lax.axis_index("s") * NC + lax.axis_index("c")
        base = wid * SPW
        pltpu.sync_copy(idxt_hbm.at[:, pl.ds(base, SPW)], idx_v)
        gsems = (gsem0, gsem1)

        def split(g):
            return g // halves, (g % halves) * BLK

        def start_gather(g, s, b):
            h, off = split(g)
            pltpu.async_copy(
                table_hbm.at[idx_v.at[h, pl.ds(off, BLK)]], rows_v.at[s, b], gsems[s])

        def start_store(g, s, b):
            h, off = split(g)
            pltpu.async_copy(
                rows_v.at[s, b], out_hbm.at[h, pl.ds(base + off, BLK)], ssem)

        def wait_one(sem, s, b):
            pltpu.make_async_copy(rows_v.at[s, b], out_hbm.at[0, pl.ds(base, BLK)], sem).wait()

        def window(w, s, first, last):
            def drain_prev():
                for b in range(K):
                    wait_one(ssem, 1 - s, b)
            if first is None:
                drain_prev()
            else:
                pl.when(jnp.logical_not(first))(drain_prev)

            def next_gathers():
                for b in range(K):
                    start_gather((w + 1) * K + b, 1 - s, b)
            if last is None:
                next_gathers()
            else:
                pl.when(jnp.logical_not(last))(next_gathers)

            for b in range(K):
                wait_one(gsems[s], s, b)
            for b in range(K):
                start_store(w * K + b, s, b)

        for b in range(K):
            start_gather(b, 0, b)

        def pair(p, carry):
            window(2 * p, 0, first=(p == 0), last=None)
            window(2 * p + 1, 1, first=None, last=(p == npair - 1))
            return carry

        lax.fori_loop(0, npair, pair, 0)

        for b in range(K):
            wait_one(ssem, 1, b)

    return gather


def kernel(x, table):
    b, h = x.shape
    out_t = _make_gather(b, h, table.shape[1])(table, jnp.swapaxes(x, 0, 1))
    return jnp.transpose(out_t, (1, 0, 2))


# speedup vs baseline: 1.0028x; 1.0028x over previous
import functools

import jax
import jax.numpy as jnp
from jax import lax
from jax.experimental import pallas as pl
from jax.experimental.pallas import tpu as pltpu
from jax.experimental.pallas import tpu_sc as plsc

NC, NS = 2, 16
NW = NC * NS
SPW = 128
BLK = 32
K = 10


@functools.lru_cache(maxsize=None)
def _make_gather(batch: int, hist: int, d_model: int):
    assert batch == NW * SPW
    halves = SPW // BLK
    ng = hist * halves
    nwin = ng // K
    npair = nwin // 2
    assert ng % K == 0 and nwin % 2 == 0

    mesh = plsc.VectorSubcoreMesh(core_axis_name="c", subcore_axis_name="s")

    @functools.partial(
        pl.kernel,
        mesh=mesh,
        compiler_params=pltpu.CompilerParams(
            disable_bounds_checks=True,
            disable_semaphore_checks=True,
        ),
        out_type=jax.ShapeDtypeStruct((hist, batch, d_model), jnp.float32),
        scratch_types=[
            pltpu.VMEM((hist, SPW), jnp.int32),
            pltpu.VMEM((2, K, BLK, d_model), jnp.float32),
            pltpu.SemaphoreType.DMA,
            pltpu.SemaphoreType.DMA,
            pltpu.SemaphoreType.DMA,
        ],
    )
    def gather(table_hbm, idxt_hbm, out_hbm, idx_v, rows_v, gsem0, gsem1, ssem):
        wid = lax.axis_index("s") * NC + lax.axis_index("c")
        base = wid * SPW
        pltpu.sync_copy(idxt_hbm.at[:, pl.ds(base, SPW)], idx_v)
        gsems = (gsem0, gsem1)

        def split(g):
            return g // halves, (g % halves) * BLK

        def start_gather(g, s, b):
            h, off = split(g)
            pltpu.async_copy(
                table_hbm.at[idx_v.at[h, pl.ds(off, BLK)]], rows_v.at[s, b], gsems[s])

        def start_store(g, s, b):
            h, off = split(g)
            pltpu.async_copy(
                rows_v.at[s, b], out_hbm.at[h, pl.ds(base + off, BLK)], ssem)

        def wait_one(sem, s, b):
            pltpu.make_async_copy(rows_v.at[s, b], out_hbm.at[0, pl.ds(base, BLK)], sem).wait()

        def window(w, s, first, last):
            def drain_prev():
                for b in range(K):
                    wait_one(ssem, 1 - s, b)
            if first is None:
                drain_prev()
            else:
                pl.when(jnp.logical_not(first))(drain_prev)

            def next_gathers():
                for b in range(K):
                    start_gather((w + 1) * K + b, 1 - s, b)
            if last is None:
                next_gathers()
            else:
                pl.when(jnp.logical_not(last))(next_gathers)

            for b in range(K):
                wait_one(gsems[s], s, b)
            for b in range(K):
                start_store(w * K + b, s, b)

        for b in range(K):
            start_gather(b, 0, b)

        def pair(p, carry):
            window(2 * p, 0, first=(p == 0), last=None)
            window(2 * p + 1, 1, first=None, last=(p == npair - 1))
            return carry

        lax.fori_loop(0, npair, pair, 0)

        for b in range(K):
            wait_one(ssem, 1, b)

    return gather


def kernel(x, table):
    b, h = x.shape
    out_t = _make_gather(b, h, table.shape[1])(table, jnp.swapaxes(x, 0, 1))
    return jnp.transpose(out_t, (1, 0, 2))


# speedup vs baseline: 1.0079x; 1.0050x over previous
import functools

import jax
import jax.numpy as jnp
from jax import lax
from jax.experimental import pallas as pl
from jax.experimental.pallas import tpu as pltpu
from jax.experimental.pallas import tpu_sc as plsc

NC, NS = 2, 16
NW = NC * NS
SPW = 128
BLK = 64
K = 5


@functools.lru_cache(maxsize=None)
def _make_gather(batch: int, hist: int, d_model: int):
    assert batch == NW * SPW
    halves = SPW // BLK
    ng = hist * halves
    nwin = ng // K
    npair = nwin // 2
    assert ng % K == 0 and nwin % 2 == 0

    mesh = plsc.VectorSubcoreMesh(core_axis_name="c", subcore_axis_name="s")

    @functools.partial(
        pl.kernel,
        mesh=mesh,
        compiler_params=pltpu.CompilerParams(
            disable_bounds_checks=True,
            disable_semaphore_checks=True,
        ),
        out_type=jax.ShapeDtypeStruct((hist, batch, d_model), jnp.float32),
        scratch_types=[
            pltpu.VMEM((hist, SPW), jnp.int32),
            pltpu.VMEM((2, K, BLK, d_model), jnp.float32),
            pltpu.SemaphoreType.DMA,
            pltpu.SemaphoreType.DMA,
            pltpu.SemaphoreType.DMA,
        ],
    )
    def gather(table_hbm, idxt_hbm, out_hbm, idx_v, rows_v, gsem0, gsem1, ssem):
        wid = lax.axis_index("s") * NC + lax.axis_index("c")
        base = wid * SPW
        pltpu.sync_copy(idxt_hbm.at[:, pl.ds(base, SPW)], idx_v)
        gsems = (gsem0, gsem1)

        def split(g):
            return g // halves, (g % halves) * BLK

        def start_gather(g, s, b):
            h, off = split(g)
            pltpu.async_copy(
                table_hbm.at[idx_v.at[h, pl.ds(off, BLK)]], rows_v.at[s, b], gsems[s])

        def start_store(g, s, b):
            h, off = split(g)
            pltpu.async_copy(
                rows_v.at[s, b], out_hbm.at[h, pl.ds(base + off, BLK)], ssem)

        def wait_one(sem, s, b):
            pltpu.make_async_copy(rows_v.at[s, b], out_hbm.at[0, pl.ds(base, BLK)], sem).wait()

        def window(w, s, first, last):
            def drain_prev():
                for b in range(K):
                    wait_one(ssem, 1 - s, b)
            if first is None:
                drain_prev()
            else:
                pl.when(jnp.logical_not(first))(drain_prev)

            def next_gathers():
                for b in range(K):
                    start_gather((w + 1) * K + b, 1 - s, b)
            if last is None:
                next_gathers()
            else:
                pl.when(jnp.logical_not(last))(next_gathers)

            for b in range(K):
                wait_one(gsems[s], s, b)
            for b in range(K):
                start_store(w * K + b, s, b)

        for b in range(K):
            start_gather(b, 0, b)

        def pair(p, carry):
            window(2 * p, 0, first=(p == 0), last=None)
            window(2 * p + 1, 1, first=None, last=(p == npair - 1))
            return carry

        lax.fori_loop(0, npair, pair, 0)

        for b in range(K):
            wait_one(ssem, 1, b)

    return gather


def kernel(x, table):
    b, h = x.shape
    out_t = _make_gather(b, h, table.shape[1])(table, jnp.swapaxes(x, 0, 1))
    return jnp.transpose(out_t, (1, 0, 2))
